# Initial kernel scaffold; baseline (speedup 1.0000x reference)
import jax
import jax.numpy as jnp
from jax.experimental import pallas as pl


def kernel(msg, index, t):
    raise NotImplementedError("write your pallas kernel here")


# speedup vs baseline: 6.2273x; 6.2273x over previous
import functools

import jax
import jax.numpy as jnp
from jax import lax
from jax.experimental import pallas as pl
from jax.experimental.pallas import tpu as pltpu
from jax.experimental.pallas import tpu_sc as plsc

_N, _D, _M = 320000, 128, 10000
_L = 16
_NT = 16
_EV = _N // _NT
_MP = 10240
_CS = _MP // _NT
_MAIN = 624
_CH = 104


def _scatter_max(tab, idx, val):
    cur = plsc.load_gather(tab, [idx])
    upd = val > cur
    plsc.store_scatter(tab, [idx], val, mask=upd)
    back = plsc.load_gather(tab, [idx], mask=upd)
    bad = jnp.sum((upd & (back != val)).astype(jnp.int32)) > 0

    @pl.when(bad)
    def _repair():
        def rbody(r, _):
            c = plsc.load_gather(tab, [idx])
            u = val > c
            plsc.store_scatter(tab, [idx], val, mask=u)
            return 0

        lax.fori_loop(0, _L - 1, rbody, 0)


def _merge_tables(tab, stage, merged, sid, accb, inb):
    pltpu.sync_copy(tab, stage.at[pl.ds(pl.multiple_of(sid * _MP, 8), _MP)])
    plsc.subcore_barrier()
    col = sid * _CS
    pltpu.sync_copy(stage.at[pl.ds(pl.multiple_of(col, 8), _CS)], accb)

    def rbody(r, _):
        off = pl.multiple_of(r * _MP + col, 8)
        pltpu.sync_copy(stage.at[pl.ds(off, _CS)], inb)

        def ubody(u, _):
            sl = pl.ds(u * _L, _L)
            accb[sl] = jnp.maximum(accb[sl], inb[sl])
            return 0

        lax.fori_loop(0, _CS // _L, ubody, 0)
        return 0

    lax.fori_loop(1, _NT, rbody, 0)
    pltpu.sync_copy(accb, merged.at[pl.ds(pl.multiple_of(col, 8), _CS)])
    plsc.subcore_barrier()
    pltpu.sync_copy(merged, tab)


def _build_kernel():
    mesh = plsc.VectorSubcoreMesh(core_axis_name="c", subcore_axis_name="s")

    @functools.partial(
        pl.kernel,
        out_type=[
            jax.ShapeDtypeStruct((_M,), jnp.int32),
            jax.ShapeDtypeStruct((_M, _D), jnp.float32),
        ],
        mesh=mesh,
        compiler_params=pltpu.CompilerParams(needs_layout_passes=False),
        scratch_types=[
            pltpu.VMEM((_EV,), jnp.int32),
            pltpu.VMEM((_EV,), jnp.int32),
            pltpu.VMEM((_MP,), jnp.int32),
            pltpu.VMEM((_MP,), jnp.int32),
            pltpu.VMEM((_MP,), jnp.int32),
            pltpu.VMEM((_MP,), jnp.int32),
            pltpu.VMEM((_CS,), jnp.int32),
            pltpu.VMEM((_CS,), jnp.int32),
            pltpu.VMEM((_CH,), jnp.int32),
            pltpu.VMEM((_L,), jnp.int32),
            pltpu.VMEM((_CH, _D), jnp.float32),
            pltpu.VMEM_SHARED((_NT * _MP,), jnp.int32),
            pltpu.VMEM_SHARED((_MP,), jnp.int32),
            pltpu.VMEM_SHARED((_MP,), jnp.int32),
            pltpu.SemaphoreType.DMA,
        ],
    )
    def lastagg(msg_hbm, idx_hbm, t_hbm, uniq_hbm, out_hbm,
                ev_idx, ev_t, maxt, argt, uniqv, srcv, accb, inb,
                idxb, idxb2, rowb, stage, merged, srows, sem):
        cid = lax.axis_index("c")
        sid = lax.axis_index("s")

        @pl.when(cid == 0)
        def _core0():
            lane = lax.iota(jnp.int32, _L)
            neg1 = jnp.full((_L,), -1, jnp.int32)

            def ibody(u, _):
                sl = pl.ds(u * _L, _L)
                maxt[sl] = neg1
                argt[sl] = neg1
                return 0

            lax.fori_loop(0, _MP // _L, ibody, 0)

            base = pl.multiple_of(sid * _EV, 8)
            pltpu.sync_copy(idx_hbm.at[pl.ds(base, _EV)], ev_idx)
            pltpu.sync_copy(t_hbm.at[pl.ds(base, _EV)], ev_t)

            def p1(v, _):
                sl = pl.ds(v * _L, _L)
                _scatter_max(maxt, ev_idx[sl], ev_t[sl])
                return 0

            lax.fori_loop(0, _EV // _L, p1, 0)
            _merge_tables(maxt, stage, merged, sid, accb, inb)

            def p2(v, _):
                sl = pl.ds(v * _L, _L)
                idx = ev_idx[sl]
                tv = ev_t[sl]
                gm = plsc.load_gather(maxt, [idx])
                gid = jnp.full((_L,), sid * _EV + v * _L, jnp.int32) + lane
                cand = jnp.where(tv == gm, gid, jnp.full((_L,), -1, jnp.int32))
                _scatter_max(argt, idx, cand)
                return 0

            lax.fori_loop(0, _EV // _L, p2, 0)
            _merge_tables(argt, stage, merged, sid, accb, inb)

            @pl.when(sid == 0)
            def _compact():
                def cbody(v, off):
                    sl = pl.ds(v * _L, _L)
                    pres = maxt[sl] >= 0
                    ids = jnp.full((_L,), v * _L, jnp.int32) + lane
                    plsc.store_compressed(uniqv.at[pl.ds(off, _L)], ids,
                                          mask=pres)
                    plsc.store_compressed(srcv.at[pl.ds(off, _L)], argt[sl],
                                          mask=pres)
                    return off + jnp.sum(pres.astype(jnp.int32))

                kcnt = lax.fori_loop(0, _MP // _L, cbody, jnp.int32(0))

                z16 = jnp.zeros((_L,), jnp.int32)
                fill_u = plsc.load_gather(uniqv, [z16])
                fill_s = plsc.load_gather(srcv, [z16])

                def fbody(v, _):
                    sl = pl.ds(v * _L, _L)
                    pos = jnp.full((_L,), v * _L, jnp.int32) + lane
                    tail = pos >= kcnt
                    uniqv[sl] = jnp.where(tail, fill_u, uniqv[sl])
                    srcv[sl] = jnp.where(tail, fill_s, srcv[sl])
                    return 0

                lax.fori_loop(kcnt // _L, _MP // _L, fbody, 0)
                pltpu.sync_copy(uniqv.at[pl.ds(0, _M)], uniq_hbm)
                pltpu.sync_copy(srcv, srows)

            plsc.subcore_barrier()

            def gbody(k, _):
                off = pl.multiple_of(sid * _MAIN + k * _CH, 8)
                pltpu.sync_copy(srows.at[pl.ds(off, _CH)], idxb)
                pltpu.async_copy(msg_hbm.at[idxb], rowb, sem).wait()
                pltpu.sync_copy(rowb, out_hbm.at[pl.ds(off, _CH)])
                return 0

            lax.fori_loop(0, _MAIN // _CH, gbody, 0)

            @pl.when(sid == _NT - 1)
            def _tail():
                toff = pl.multiple_of(_NT * _MAIN, 8)
                pltpu.sync_copy(srows.at[pl.ds(toff, _L)], idxb2)
                pltpu.async_copy(msg_hbm.at[idxb2], rowb.at[pl.ds(0, _L)],
                                 sem).wait()
                pltpu.sync_copy(rowb.at[pl.ds(0, _L)],
                                out_hbm.at[pl.ds(toff, _L)])

    return lastagg


_lastagg = _build_kernel()


@jax.jit
def kernel(msg, index, t):
    uniq, rows = _lastagg(msg, index, t)
    return uniq, rows
